# baseline (speedup 1.0000x reference)
import jax
import jax.numpy as jnp
from jax import lax
from jax.experimental import pallas as pl
from jax.experimental.pallas import tpu as pltpu
from jax.experimental.pallas import tpu_sc as plsc

D = 1024
K = 1024
LV = 4
NS = 16
L = 16
RCH = 16
DB = D // L
ROWS = K // NS
NCH = ROWS // RCH
HR = 32
HCH = HR // RCH
BIG = 3.0e38
BIGI = 2 ** 30


def _row_sums(slab, c, vec, dist):
    def body(db, accs):
        ds = pl.ds(db * L, L)
        vblk = vec[ds]
        out = []
        for row in range(RCH):
            blk = slab[c * RCH + row, ds]
            if dist:
                dd = vblk - blk
                out.append(accs[row] + dd * dd)
            else:
                out.append(accs[row] + vblk * blk)
        return tuple(out)

    zero = jnp.zeros((L,), jnp.float32)
    accs = lax.fori_loop(0, DB, body, (zero,) * RCH)
    lane = lax.iota(jnp.int32, L)
    sums = jnp.zeros((L,), jnp.float32)
    for row in range(RCH):
        s = jnp.sum(accs[row])
        sums = jnp.where(lane == row, jnp.full((L,), s), sums)
    return sums


def _rvq_sc_body(x_hbm, W_hbm, C_hbm, qout_hbm, idx_hbm, loss_hbm,
                 sl0, sl1, sl2, xv, rv, qv, pv, plocal, vrow, av_all,
                 stage_f, stage_i, stage_p, psh, avsh,
                 sm0, sm1, sm2):
    sid = lax.axis_index("s")
    lane = lax.iota(jnp.int32, L)
    slabs = (sl0, sl1, sl2)
    sems = (sm0, sm1, sm2)

    seq = []
    for i in range(LV):
        seq.append((W_hbm, i, 0))
        seq.append((W_hbm, i, 1))
        seq.append((C_hbm, i, 0))
        seq.append((C_hbm, i, 1))

    HH = HR // 2

    def start(n):
        src, i, h = seq[n]
        row0 = i * K + sid * ROWS + h * HR
        s = n % 3
        c0 = pltpu.async_copy(src.at[pl.ds(row0, HH)],
                              slabs[s].at[pl.ds(0, HH)], sems[s])
        c1 = pltpu.async_copy(src.at[pl.ds(row0 + HH, HH)],
                              slabs[s].at[pl.ds(HH, HH)], sems[s])
        return (c0, c1)

    copies = {}
    for n in range(3):
        copies[n] = start(n)

    pltpu.sync_copy(x_hbm, xv)

    def init_body(db, c):
        ds = pl.ds(db * L, L)
        rv[ds] = xv[ds]
        qv[ds] = jnp.zeros((L,), jnp.float32)
        return c

    lax.fori_loop(0, DB, init_body, 0)

    idx_vec = jnp.zeros((L,), jnp.int32)
    loss_vec = jnp.zeros((L,), jnp.float32)

    for i in range(LV):
        slot = i % 2

        for h in range(2):
            n = 4 * i + h
            slab = slabs[n % 3]
            copies[n][0].wait()
            copies[n][1].wait()

            def w_body(c, carry, slab=slab, h=h):
                sums = _row_sums(slab, c, rv, dist=False)
                plocal[pl.ds(h * HR + c * RCH, RCH)] = sums
                return carry

            lax.fori_loop(0, HCH, w_body, 0)
            if n + 3 < len(seq):
                copies[n + 3] = start(n + 3)

        pltpu.sync_copy(plocal, psh.at[slot, pl.ds(sid * ROWS, ROWS)])
        plsc.subcore_barrier()
        pltpu.sync_copy(psh.at[slot], pv)

        carry = (jnp.float32(BIG), jnp.int32(BIGI))
        for h in range(2):
            n = 4 * i + 2 + h
            slab = slabs[n % 3]
            copies[n][0].wait()
            copies[n][1].wait()

            def c_body(c, carry, slab=slab, h=h):
                best_val, best_idx = carry
                s16 = _row_sums(slab, c, pv, dist=True)
                row_ids = jnp.full((L,), sid * ROWS + h * HR + c * RCH) + lane
                cmin = jnp.min(s16)
                cidx = jnp.min(jnp.where(s16 == cmin, row_ids,
                                         jnp.int32(BIGI)))
                take = cmin < best_val
                best_idx = jnp.where(take, cidx, best_idx)
                best_val = jnp.where(take, cmin, best_val)
                return best_val, best_idx

            carry = lax.fori_loop(0, HCH, c_body, carry)
            if n + 3 < len(seq):
                copies[n + 3] = start(n + 3)
        best_val, best_idx = carry

        stage_p[0, :] = jnp.full((L,), best_val)
        stage_p[1, :] = plsc.bitcast(jnp.full((L,), best_idx), jnp.float32)
        stage_p[2, :] = jnp.zeros((L,), jnp.float32)
        stage_p[3, :] = jnp.zeros((L,), jnp.float32)
        pltpu.sync_copy(stage_p, avsh.at[slot, sid])
        plsc.subcore_barrier()
        pltpu.sync_copy(avsh.at[slot], av_all)
        gval = jnp.float32(BIG)
        gidx = jnp.int32(0)
        for t in range(NS):
            v_t = jnp.min(av_all[t, 0])
            i_t = jnp.min(plsc.bitcast(av_all[t, 1], jnp.int32))
            take = v_t < gval
            gidx = jnp.where(take, i_t, gidx)
            gval = jnp.where(take, v_t, gval)

        gidx = jnp.clip(gidx, 0, K - 1)
        pltpu.sync_copy(C_hbm.at[pl.ds(i * K + gidx, 1)], vrow)

        def upd_body(db, lacc):
            ds = pl.ds(db * L, L)
            vblk = vrow[0, ds]
            rv[ds] = rv[ds] - vblk
            qv[ds] = qv[ds] + vblk
            dx = vblk - xv[ds]
            return lacc + dx * dx

        lacc = lax.fori_loop(0, DB, upd_body,
                             jnp.zeros((L,), jnp.float32))
        loss_i = jnp.sum(lacc) * jnp.float32(1.0 / D)
        idx_vec = jnp.where(lane == i, jnp.full((L,), gidx), idx_vec)
        loss_vec = jnp.where(lane == i, jnp.full((L,), loss_i), loss_vec)

    def recon_body(db, racc):
        ds = pl.ds(db * L, L)
        dq = qv[ds] - xv[ds]
        return racc + dq * dq

    racc = lax.fori_loop(0, DB, recon_body, jnp.zeros((L,), jnp.float32))
    recon = jnp.sum(racc) * jnp.float32(1.0 / D)
    loss_vec = jnp.where(lane == LV, jnp.full((L,), recon), loss_vec)

    @pl.when(sid == 0)
    def _write():
        pltpu.sync_copy(qv, qout_hbm)
        stage_i[...] = idx_vec
        pltpu.sync_copy(stage_i, idx_hbm)
        stage_f[...] = loss_vec
        pltpu.sync_copy(stage_f, loss_hbm)


def kernel(x, W, C):
    mesh = plsc.VectorSubcoreMesh(core_axis_name="c", subcore_axis_name="s",
                                  num_cores=2, num_subcores=NS)
    run = pl.kernel(
        _rvq_sc_body,
        out_type=[
            jax.ShapeDtypeStruct((D,), jnp.float32),
            jax.ShapeDtypeStruct((L,), jnp.int32),
            jax.ShapeDtypeStruct((L,), jnp.float32),
        ],
        mesh=mesh,
        compiler_params=pltpu.CompilerParams(needs_layout_passes=False),
        scratch_types=[
            pltpu.VMEM((HR, D), jnp.float32),
            pltpu.VMEM((HR, D), jnp.float32),
            pltpu.VMEM((HR, D), jnp.float32),
            pltpu.VMEM((D,), jnp.float32),
            pltpu.VMEM((D,), jnp.float32),
            pltpu.VMEM((D,), jnp.float32),
            pltpu.VMEM((D,), jnp.float32),
            pltpu.VMEM((ROWS,), jnp.float32),
            pltpu.VMEM((1, D), jnp.float32),
            pltpu.VMEM((NS, 4, L), jnp.float32),
            pltpu.VMEM((L,), jnp.float32),
            pltpu.VMEM((L,), jnp.int32),
            pltpu.VMEM((4, L), jnp.float32),
            pltpu.VMEM_SHARED((2, K), jnp.float32),
            pltpu.VMEM_SHARED((2, NS, 4, L), jnp.float32),
            pltpu.SemaphoreType.DMA,
            pltpu.SemaphoreType.DMA,
            pltpu.SemaphoreType.DMA,
        ],
    )
    qout, idxo, losso = run(x.reshape(D), W.reshape(LV * K, D),
                            C.reshape(LV * K, D))
    return qout.reshape(1, D), idxo[:LV], losso[:LV + 1]


# speedup vs baseline: 1.0198x; 1.0198x over previous
import jax
import jax.numpy as jnp
from jax import lax
from jax.experimental import pallas as pl
from jax.experimental.pallas import tpu as pltpu
from jax.experimental.pallas import tpu_sc as plsc

D = 1024
K = 1024
LV = 4
NS = 16
L = 16
RCH = 16
DB = D // L
ROWS = K // NS
NCH = ROWS // RCH
HR = 32
HCH = HR // RCH
BIG = 3.0e38
BIGI = 2 ** 30


def _row_sums(slab, c, vec, dist):
    def body(db, accs):
        ds = pl.ds(db * L, L)
        vblk = vec[ds]
        out = []
        for row in range(RCH):
            blk = slab[c * RCH + row, ds]
            if dist:
                dd = vblk - blk
                out.append(accs[row] + dd * dd)
            else:
                out.append(accs[row] + vblk * blk)
        return tuple(out)

    zero = jnp.zeros((L,), jnp.float32)
    accs = lax.fori_loop(0, DB, body, (zero,) * RCH)
    lane = lax.iota(jnp.int32, L)
    sums = jnp.zeros((L,), jnp.float32)
    for row in range(RCH):
        s = jnp.sum(accs[row])
        sums = jnp.where(lane == row, jnp.full((L,), s), sums)
    return sums


def _rvq_sc_body(x_hbm, W_hbm, C_hbm, qout_hbm, idx_hbm, loss_hbm, exch_hbm,
                 sl0, sl1, sl2, xv, rv, qv, pv, plocal, vrow, av_all,
                 stage_f, stage_i, stage_p, xbuf, psh, avsh, gsh,
                 sm0, sm1, sm2, xsem):
    sid = lax.axis_index("s")
    cid = lax.axis_index("c")
    lane = lax.iota(jnp.int32, L)
    slabs = (sl0, sl1, sl2)
    sems = (sm0, sm1, sm2)

    seq = []
    for i in range(LV):
        seq.append(("W", i, 0))
        seq.append(("W", i, 1))
        seq.append(("C", i, 0))

    HH = HR // 2

    def start(n):
        kind, i, h = seq[n]
        if kind == "W":
            src = W_hbm
            row0 = i * K + sid * ROWS + h * HR
        else:
            src = C_hbm
            row0 = i * K + cid * (K // 2) + sid * HR
        s = n % 3
        c0 = pltpu.async_copy(src.at[pl.ds(row0, HH)],
                              slabs[s].at[pl.ds(0, HH)], sems[s])
        c1 = pltpu.async_copy(src.at[pl.ds(row0 + HH, HH)],
                              slabs[s].at[pl.ds(HH, HH)], sems[s])
        return (c0, c1)

    copies = {}
    for n in range(3):
        copies[n] = start(n)

    pltpu.sync_copy(x_hbm, xv)

    def init_body(db, c):
        ds = pl.ds(db * L, L)
        rv[ds] = xv[ds]
        qv[ds] = jnp.zeros((L,), jnp.float32)
        return c

    lax.fori_loop(0, DB, init_body, 0)

    idx_vec = jnp.zeros((L,), jnp.int32)
    loss_vec = jnp.zeros((L,), jnp.float32)

    for i in range(LV):
        slot = i % 2

        for h in range(2):
            n = 3 * i + h
            slab = slabs[n % 3]
            copies[n][0].wait()
            copies[n][1].wait()

            def w_body(c, carry, slab=slab, h=h):
                sums = _row_sums(slab, c, rv, dist=False)
                plocal[pl.ds(h * HR + c * RCH, RCH)] = sums
                return carry

            lax.fori_loop(0, HCH, w_body, 0)
            if n + 3 < len(seq):
                copies[n + 3] = start(n + 3)

        pltpu.sync_copy(plocal, psh.at[slot, pl.ds(sid * ROWS, ROWS)])
        plsc.subcore_barrier()
        pltpu.sync_copy(psh.at[slot], pv)

        carry = (jnp.float32(BIG), jnp.int32(BIGI))
        for h in range(1):
            n = 3 * i + 2
            slab = slabs[n % 3]
            copies[n][0].wait()
            copies[n][1].wait()
            base = cid * (K // 2) + sid * HR

            def c_body(c, carry, slab=slab, base=base):
                best_val, best_idx = carry
                s16 = _row_sums(slab, c, pv, dist=True)
                row_ids = jnp.full((L,), base + c * RCH) + lane
                cmin = jnp.min(s16)
                cidx = jnp.min(jnp.where(s16 == cmin, row_ids,
                                         jnp.int32(BIGI)))
                take = cmin < best_val
                best_idx = jnp.where(take, cidx, best_idx)
                best_val = jnp.where(take, cmin, best_val)
                return best_val, best_idx

            carry = lax.fori_loop(0, HCH, c_body, carry)
            if n + 3 < len(seq):
                copies[n + 3] = start(n + 3)
        best_val, best_idx = carry

        stage_p[0, :] = jnp.full((L,), best_val)
        stage_p[1, :] = plsc.bitcast(jnp.full((L,), best_idx), jnp.float32)
        stage_p[2, :] = jnp.zeros((L,), jnp.float32)
        stage_p[3, :] = jnp.zeros((L,), jnp.float32)
        pltpu.sync_copy(stage_p, avsh.at[slot, sid])
        plsc.subcore_barrier()
        pltpu.sync_copy(avsh.at[slot], av_all)
        gval = jnp.float32(BIG)
        gidx = jnp.int32(0)
        for t in range(NS):
            v_t = jnp.min(av_all[t, 0])
            i_t = jnp.min(plsc.bitcast(av_all[t, 1], jnp.int32))
            take = v_t < gval
            gidx = jnp.where(take, i_t, gidx)
            gval = jnp.where(take, v_t, gval)

        @pl.when(sid == 0)
        def _exch():
            xbuf[0, :] = jnp.full((L,), gval)
            xbuf[1, :] = plsc.bitcast(jnp.full((L,), gidx), jnp.float32)
            xbuf[2, :] = jnp.zeros((L,), jnp.float32)
            xbuf[3, :] = jnp.zeros((L,), jnp.float32)
            pltpu.sync_copy(xbuf, exch_hbm.at[2 * i + cid])
            pltpu.sync_copy(exch_hbm.at[2 * i + cid], xbuf)
            pltpu.semaphore_signal(
                xsem, 1, device_id={"c": 1 - cid, "s": 0})
            pltpu.semaphore_wait(xsem, 1)
            pltpu.sync_copy(exch_hbm.at[2 * i + (1 - cid)], xbuf)
            oval = jnp.min(xbuf[0])
            oidx = jnp.min(plsc.bitcast(xbuf[1], jnp.int32))
            take_o = jnp.logical_or(
                oval < gval,
                jnp.logical_and(oval == gval, oidx < gidx))
            g2val = jnp.where(take_o, oval, gval)
            g2idx = jnp.where(take_o, oidx, gidx)
            stage_p[0, :] = jnp.full((L,), g2val)
            stage_p[1, :] = plsc.bitcast(jnp.full((L,), g2idx), jnp.float32)
            pltpu.sync_copy(stage_p, gsh.at[slot])
        plsc.subcore_barrier()
        pltpu.sync_copy(gsh.at[slot], stage_p)
        gidx = jnp.min(plsc.bitcast(stage_p[1, :], jnp.int32))

        gidx = jnp.clip(gidx, 0, K - 1)
        pltpu.sync_copy(C_hbm.at[pl.ds(i * K + gidx, 1)], vrow)

        def upd_body(db, lacc):
            ds = pl.ds(db * L, L)
            vblk = vrow[0, ds]
            rv[ds] = rv[ds] - vblk
            qv[ds] = qv[ds] + vblk
            dx = vblk - xv[ds]
            return lacc + dx * dx

        lacc = lax.fori_loop(0, DB, upd_body,
                             jnp.zeros((L,), jnp.float32))
        loss_i = jnp.sum(lacc) * jnp.float32(1.0 / D)
        idx_vec = jnp.where(lane == i, jnp.full((L,), gidx), idx_vec)
        loss_vec = jnp.where(lane == i, jnp.full((L,), loss_i), loss_vec)

    def recon_body(db, racc):
        ds = pl.ds(db * L, L)
        dq = qv[ds] - xv[ds]
        return racc + dq * dq

    racc = lax.fori_loop(0, DB, recon_body, jnp.zeros((L,), jnp.float32))
    recon = jnp.sum(racc) * jnp.float32(1.0 / D)
    loss_vec = jnp.where(lane == LV, jnp.full((L,), recon), loss_vec)

    @pl.when(sid == 0)
    def _write():
        pltpu.sync_copy(qv, qout_hbm)
        stage_i[...] = idx_vec
        pltpu.sync_copy(stage_i, idx_hbm)
        stage_f[...] = loss_vec
        pltpu.sync_copy(stage_f, loss_hbm)


def kernel(x, W, C):
    mesh = plsc.VectorSubcoreMesh(core_axis_name="c", subcore_axis_name="s",
                                  num_cores=2, num_subcores=NS)
    run = pl.kernel(
        _rvq_sc_body,
        out_type=[
            jax.ShapeDtypeStruct((D,), jnp.float32),
            jax.ShapeDtypeStruct((L,), jnp.int32),
            jax.ShapeDtypeStruct((L,), jnp.float32),
            jax.ShapeDtypeStruct((2 * LV, 4, L), jnp.float32),
        ],
        mesh=mesh,
        compiler_params=pltpu.CompilerParams(needs_layout_passes=False),
        scratch_types=[
            pltpu.VMEM((HR, D), jnp.float32),
            pltpu.VMEM((HR, D), jnp.float32),
            pltpu.VMEM((HR, D), jnp.float32),
            pltpu.VMEM((D,), jnp.float32),
            pltpu.VMEM((D,), jnp.float32),
            pltpu.VMEM((D,), jnp.float32),
            pltpu.VMEM((D,), jnp.float32),
            pltpu.VMEM((ROWS,), jnp.float32),
            pltpu.VMEM((1, D), jnp.float32),
            pltpu.VMEM((NS, 4, L), jnp.float32),
            pltpu.VMEM((L,), jnp.float32),
            pltpu.VMEM((L,), jnp.int32),
            pltpu.VMEM((4, L), jnp.float32),
            pltpu.VMEM((4, L), jnp.float32),
            pltpu.VMEM_SHARED((2, K), jnp.float32),
            pltpu.VMEM_SHARED((2, NS, 4, L), jnp.float32),
            pltpu.VMEM_SHARED((2, 4, L), jnp.float32),
            pltpu.SemaphoreType.DMA,
            pltpu.SemaphoreType.DMA,
            pltpu.SemaphoreType.DMA,
            pltpu.SemaphoreType.REGULAR,
        ],
    )
    qout, idxo, losso, _ = run(x.reshape(D), W.reshape(LV * K, D),
                               C.reshape(LV * K, D))
    return qout.reshape(1, D), idxo[:LV], losso[:LV + 1]
